# Initial kernel scaffold; baseline (speedup 1.0000x reference)
import jax
import jax.numpy as jnp
from jax.experimental import pallas as pl


def kernel(x, table, W, b):
    raise NotImplementedError("write your pallas kernel here")


# speedup vs baseline: 8.0983x; 8.0983x over previous
import functools

import jax
import jax.numpy as jnp
from jax import lax
from jax.experimental import pallas as pl
from jax.experimental.pallas import tpu as pltpu
from jax.experimental.pallas import tpu_sc as plsc

_VOCAB = 100000
_D = 64
_B = 4096
_H = 200
_HH = _H // 2
_C = 5
_DP = 16

_NC = 2
_NS = 16
_NW = _NC * _NS
_BPW = _B // _NW

_ROWBLK = 4000


def _proj_body(t_ref, w_ref, o_ref):
    o_ref[...] = jnp.dot(t_ref[...], w_ref[...],
                         preferred_element_type=jnp.float32)


def _project(table, wp):
    return pl.pallas_call(
        _proj_body,
        grid=(_VOCAB // _ROWBLK,),
        in_specs=[
            pl.BlockSpec((_ROWBLK, _D), lambda i: (i, 0)),
            pl.BlockSpec((_D, _DP), lambda i: (0, 0)),
        ],
        out_specs=pl.BlockSpec((_ROWBLK, _DP), lambda i: (i, 0)),
        out_shape=jax.ShapeDtypeStruct((_VOCAB, _DP), jnp.float32),
    )(table, wp)


def _make_pool():
    mesh = plsc.VectorSubcoreMesh(core_axis_name="c", subcore_axis_name="s")

    @functools.partial(
        pl.kernel,
        mesh=mesh,
        out_type=jax.ShapeDtypeStruct((_B, _DP), jnp.float32),
        scratch_types=[
            pltpu.VMEM((2 * _BPW, _HH), jnp.int32),
            pltpu.VMEM((_H, _DP), jnp.float32),
            pltpu.VMEM((_BPW, _DP), jnp.float32),
            pltpu.VMEM((_DP,), jnp.float32),
            pltpu.SemaphoreType.DMA,
        ],
        compiler_params=pltpu.CompilerParams(use_tc_tiling_on_sc=False),
    )
    def pool(x2_hbm, tw_hbm, bias_hbm, out_hbm, idx_v, rows_v, out_v, bias_v,
             sem):
        wid = lax.axis_index("s") * _NC + lax.axis_index("c")
        base = wid * _BPW
        pltpu.sync_copy(x2_hbm.at[pl.ds(2 * base, 2 * _BPW)], idx_v)
        pltpu.sync_copy(bias_hbm, bias_v)
        bias = bias_v[...]

        def row(r, carry):
            pltpu.async_copy(tw_hbm.at[idx_v.at[2 * r]],
                             rows_v.at[pl.ds(0, _HH)], sem).wait()
            pltpu.async_copy(tw_hbm.at[idx_v.at[2 * r + 1]],
                             rows_v.at[pl.ds(_HH, _HH)], sem).wait()
            accs = [bias, jnp.zeros((_DP,), jnp.float32),
                    jnp.zeros((_DP,), jnp.float32),
                    jnp.zeros((_DP,), jnp.float32)]
            for j in range(_H):
                accs[j % 4] = accs[j % 4] + rows_v[j]
            out_v[r] = (accs[0] + accs[1]) + (accs[2] + accs[3])
            return carry

        lax.fori_loop(0, _BPW, row, 0)
        pltpu.sync_copy(out_v, out_hbm.at[pl.ds(base, _BPW)])

    return pool


_pool_call = _make_pool()


def kernel(x, table, W, b):
    x = x.astype(jnp.int32)
    wp = jnp.pad(W, ((0, 0), (0, _DP - _C)))
    bp = jnp.pad(b, (0, _DP - _C))
    tw = _project(table, wp)
    x2 = x.reshape(2 * _B, _HH)
    out16 = _pool_call(x2, tw, bp)
    return out16[:, :_C]


# speedup vs baseline: 15.5271x; 1.9173x over previous
import functools

import jax
import jax.numpy as jnp
from jax import lax
from jax.experimental import pallas as pl
from jax.experimental.pallas import tpu as pltpu
from jax.experimental.pallas import tpu_sc as plsc

_VOCAB = 100000
_D = 64
_B = 4096
_H = 200
_HH = _H // 2
_C = 5
_DP = 16

_NC = 2
_NS = 16
_NW = _NC * _NS
_BPW = _B // _NW

_ROWBLK = 4000


def _proj_body(t_ref, w_ref, o_ref):
    o_ref[...] = jnp.dot(t_ref[...], w_ref[...],
                         preferred_element_type=jnp.float32)


def _project(table, wp):
    return pl.pallas_call(
        _proj_body,
        grid=(_VOCAB // _ROWBLK,),
        in_specs=[
            pl.BlockSpec((_ROWBLK, _D), lambda i: (i, 0)),
            pl.BlockSpec((_D, _DP), lambda i: (0, 0)),
        ],
        out_specs=pl.BlockSpec((_ROWBLK, _DP), lambda i: (i, 0)),
        out_shape=jax.ShapeDtypeStruct((_VOCAB, _DP), jnp.float32),
    )(table, wp)


_NSLOT = 4


def _make_pool():
    mesh = plsc.VectorSubcoreMesh(core_axis_name="c", subcore_axis_name="s")

    @functools.partial(
        pl.kernel,
        mesh=mesh,
        out_type=jax.ShapeDtypeStruct((_B, _DP), jnp.float32),
        scratch_types=[
            pltpu.VMEM((2 * _BPW, _HH), jnp.int32),
            pltpu.VMEM((_NSLOT, _H, _DP), jnp.float32),
            pltpu.VMEM((_BPW, _DP), jnp.float32),
            pltpu.VMEM((_DP,), jnp.float32),
            [pltpu.SemaphoreType.DMA] * _NSLOT,
        ],
        compiler_params=pltpu.CompilerParams(use_tc_tiling_on_sc=False),
    )
    def pool(x2_hbm, tw_hbm, bias_hbm, out_hbm, idx_v, rows_v, out_v, bias_v,
             sems):
        wid = lax.axis_index("s") * _NC + lax.axis_index("c")
        base = wid * _BPW
        pltpu.sync_copy(x2_hbm.at[pl.ds(2 * base, 2 * _BPW)], idx_v)
        pltpu.sync_copy(bias_hbm, bias_v)
        bias = bias_v[...]

        def issue(r, s):
            pltpu.async_copy(tw_hbm.at[idx_v.at[2 * r]],
                             rows_v.at[s, pl.ds(0, _HH)], sems[s])
            pltpu.async_copy(tw_hbm.at[idx_v.at[2 * r + 1]],
                             rows_v.at[s, pl.ds(_HH, _HH)], sems[s])

        def drain(s):
            pltpu.make_async_copy(tw_hbm.at[idx_v.at[0]],
                                  rows_v.at[s, pl.ds(0, _HH)], sems[s]).wait()
            pltpu.make_async_copy(tw_hbm.at[idx_v.at[0]],
                                  rows_v.at[s, pl.ds(_HH, _HH)], sems[s]).wait()

        for s in range(_NSLOT):
            issue(s, s)

        def group(g, carry):
            for s in range(_NSLOT):
                r = _NSLOT * g + s
                drain(s)
                accs = [bias, jnp.zeros((_DP,), jnp.float32),
                        jnp.zeros((_DP,), jnp.float32),
                        jnp.zeros((_DP,), jnp.float32)]
                for j in range(_H):
                    accs[j % 4] = accs[j % 4] + rows_v[s, j]
                out_v[r] = (accs[0] + accs[1]) + (accs[2] + accs[3])

                @pl.when(r + _NSLOT < _BPW)
                def _():
                    issue(r + _NSLOT, s)
            return carry

        lax.fori_loop(0, _BPW // _NSLOT, group, 0)
        pltpu.sync_copy(out_v, out_hbm.at[pl.ds(base, _BPW)])

    return pool


_pool_call = _make_pool()


def kernel(x, table, W, b):
    x = x.astype(jnp.int32)
    wp = jnp.pad(W, ((0, 0), (0, _DP - _C)))
    bp = jnp.pad(b, (0, _DP - _C))
    tw = _project(table, wp)
    x2 = x.reshape(2 * _B, _HH)
    out16 = _pool_call(x2, tw, bp)
    return out16[:, :_C]
